# baseline (speedup 1.0000x reference)
import functools

import jax
import jax.numpy as jnp
from jax import lax
from jax.experimental import pallas as pl
from jax.experimental.pallas import tpu as pltpu
from jax.experimental.pallas import tpu_sc as plsc

E = 8
K = 256
D = 2048
H = 2048
B = 4
HT = 256
NH = 8
NW = 32


def _topk_loss_kernel(v_ref, w_ref, i_ref, loss_ref, c_ref):
    v = v_ref[0]
    iota_d = lax.broadcasted_iota(jnp.int32, (E, D), 1)
    iota_k = lax.broadcasted_iota(jnp.int32, (E, K), 1)

    def step(kk, carry):
        vals, idxs, work = carry
        m = jnp.max(work, axis=1, keepdims=True)
        am = jnp.min(jnp.where(work == m, iota_d, D), axis=1, keepdims=True)
        vals = jnp.where(iota_k == kk, m, vals)
        idxs = jnp.where(iota_k == kk, am, idxs)
        work = jnp.where(iota_d == am, -jnp.inf, work)
        return vals, idxs, work

    vals0 = jnp.zeros((E, K), jnp.float32)
    idxs0 = jnp.zeros((E, K), jnp.int32)
    vals, idxs, work = lax.fori_loop(0, K, step, (vals0, idxs0, v))
    w_ref[0] = vals
    i_ref[0] = idxs

    chosen = jnp.where(work == -jnp.inf, 1.0, 0.0).astype(jnp.float32)
    b = pl.program_id(0)

    @pl.when(b == 0)
    def _():
        c_ref[...] = chosen

    @pl.when(b > 0)
    def _():
        c_ref[...] = c_ref[...] + chosen

    @pl.when(b == B - 1)
    def _():
        u = c_ref[...] * (1.0 / (B * K + 1e-9)) - (1.0 / E)
        loss_ref[...] = (jnp.sum(u * u) * (1.0 / (E * D))).reshape(1, 1)


def _gather_kernel(x_ref, i_ref, out_ref):
    xb = x_ref[0]
    row = i_ref[0, 0, :].reshape(1, K)
    for dc in range(D // K):
        iota = lax.broadcasted_iota(jnp.int32, (K, K), 0) + dc * K
        p = (iota == row).astype(jnp.float32)
        part = lax.dot_general(
            xb[:, dc * K:(dc + 1) * K], p,
            (((1,), (0,)), ((), ())),
            preferred_element_type=jnp.float32,
        )
        if dc == 0:
            out_ref[0, 0] = part
        else:
            out_ref[0, 0] = out_ref[0, 0] + part


def _sc_gather(x_hbm, idx_hbm, out_hbm, idx_v, x_v, o_v):
    wid = lax.axis_index("s") * 2 + lax.axis_index("c")
    b = wid // (NW // B)
    hc = wid % (NW // B)
    rows_per_w = H * B // NW
    pltpu.sync_copy(idx_hbm.at[pl.ds(b * E * K, E * K)], idx_v)

    def chunk(c, _):
        h0 = hc * rows_per_w + c * NH
        pltpu.sync_copy(x_hbm.at[pl.ds((b * H + h0) * D, NH * D)], x_v)

        def eh(i, _):
            hl = i // E
            e = i % E
            base = hl * D
            for kc in range(K // 16):
                iv = idx_v[pl.ds(e * K + kc * 16, 16)]
                g = plsc.load_gather(x_v, [iv + base])
                o_v[pl.ds((e * NH + hl) * K + kc * 16, 16)] = g
            return 0

        lax.fori_loop(0, E * NH, eh, 0)
        for e in range(E):
            pltpu.sync_copy(o_v.at[pl.ds(e * NH * K, NH * K)],
                            out_hbm.at[pl.ds(((e * B + b) * H + h0) * K,
                                             NH * K)])
        return 0

    lax.fori_loop(0, H * B // NW // NH, chunk, 0)


def kernel(x, W):
    xt = jnp.swapaxes(x, -1, -2)
    router_logit = jnp.einsum('bdh,eh->bde', xt, W)
    router_logit = jax.nn.softmax(router_logit, axis=-1)
    noise = jax.random.normal(jax.random.key(1234), router_logit.shape,
                              dtype=router_logit.dtype) * 0.001
    v = router_logit + noise
    vt = jnp.swapaxes(v, 1, 2)

    w_bek, i_bek, loss = pl.pallas_call(
        _topk_loss_kernel,
        grid=(B,),
        in_specs=[pl.BlockSpec((1, E, D), lambda b: (b, 0, 0))],
        out_specs=[
            pl.BlockSpec((1, E, K), lambda b: (b, 0, 0)),
            pl.BlockSpec((1, E, K), lambda b: (b, 0, 0)),
            pl.BlockSpec((1, 1), lambda b: (0, 0)),
        ],
        out_shape=[
            jax.ShapeDtypeStruct((B, E, K), jnp.float32),
            jax.ShapeDtypeStruct((B, E, K), jnp.int32),
            jax.ShapeDtypeStruct((1, 1), jnp.float32),
        ],
        scratch_shapes=[pltpu.VMEM((E, D), jnp.float32)],
    )(vt)

    mesh = plsc.VectorSubcoreMesh(core_axis_name="c", subcore_axis_name="s")
    gather = functools.partial(
        pl.kernel,
        mesh=mesh,
        compiler_params=pltpu.CompilerParams(needs_layout_passes=False),
        out_type=jax.ShapeDtypeStruct((E * B * H * K,), jnp.float32),
        scratch_types=[
            pltpu.VMEM((E * K,), jnp.int32),
            pltpu.VMEM((NH * D,), jnp.float32),
            pltpu.VMEM((E * NH * K,), jnp.float32),
        ],
    )(_sc_gather)
    tokens = gather(x.reshape(B * H * D),
                    i_bek.reshape(B * E * K)).reshape(E, B, H, K)

    weights = jnp.transpose(w_bek, (1, 0, 2))
    indices = jnp.transpose(i_bek, (1, 0, 2))
    return tokens, weights, indices, loss.reshape(())


# speedup vs baseline: 1.4624x; 1.4624x over previous
import functools

import jax
import jax.numpy as jnp
from jax import lax
from jax.experimental import pallas as pl
from jax.experimental.pallas import tpu as pltpu
from jax.experimental.pallas import tpu_sc as plsc

E = 8
K = 256
D = 2048
H = 2048
B = 4
HT = 256
NH = 8
NW = 32


def _topk_loss_kernel(v_ref, w_ref, i_ref, loss_ref, c_ref):
    v = v_ref[0]
    iota_d = lax.broadcasted_iota(jnp.int32, (E, D), 1)
    iota_k = lax.broadcasted_iota(jnp.int32, (E, K), 1)

    def step(kk, carry):
        vals, idxs, work = carry
        m = jnp.max(work, axis=1, keepdims=True)
        am = jnp.min(jnp.where(work == m, iota_d, D), axis=1, keepdims=True)
        vals = jnp.where(iota_k == kk, m, vals)
        idxs = jnp.where(iota_k == kk, am, idxs)
        work = jnp.where(iota_d == am, -jnp.inf, work)
        return vals, idxs, work

    vals0 = jnp.zeros((E, K), jnp.float32)
    idxs0 = jnp.zeros((E, K), jnp.int32)
    vals, idxs, work = lax.fori_loop(0, K, step, (vals0, idxs0, v))
    w_ref[0] = vals
    i_ref[0] = idxs

    chosen = jnp.where(work == -jnp.inf, 1.0, 0.0).astype(jnp.float32)
    b = pl.program_id(0)

    @pl.when(b == 0)
    def _():
        c_ref[...] = chosen

    @pl.when(b > 0)
    def _():
        c_ref[...] = c_ref[...] + chosen

    @pl.when(b == B - 1)
    def _():
        u = c_ref[...] * (1.0 / (B * K + 1e-9)) - (1.0 / E)
        loss_ref[...] = (jnp.sum(u * u) * (1.0 / (E * D))).reshape(1, 1)


def _gather_kernel(x_ref, i_ref, out_ref):
    xb = x_ref[0]
    row = i_ref[0, 0, :].reshape(1, K)
    for dc in range(D // K):
        iota = lax.broadcasted_iota(jnp.int32, (K, K), 0) + dc * K
        p = (iota == row).astype(jnp.float32)
        part = lax.dot_general(
            xb[:, dc * K:(dc + 1) * K], p,
            (((1,), (0,)), ((), ())),
            preferred_element_type=jnp.float32,
        )
        if dc == 0:
            out_ref[0, 0] = part
        else:
            out_ref[0, 0] = out_ref[0, 0] + part


def _sc_gather(x_hbm, idx_hbm, out_hbm, idx_v, x_v0, x_v1, o_v0, o_v1,
               si0, si1, so0, so1):
    wid = lax.axis_index("s") * 2 + lax.axis_index("c")
    b = wid // (NW // B)
    hc = wid % (NW // B)
    rows_per_w = H * B // NW
    nchunks = rows_per_w // NH
    pltpu.sync_copy(idx_hbm.at[pl.ds(b * E * K, E * K)], idx_v)

    xv = (x_v0, x_v1)
    ov = (o_v0, o_v1)
    sin = (si0, si1)
    sout = (so0, so1)

    def xin(c, p):
        h0 = hc * rows_per_w + c * NH
        return pltpu.make_async_copy(
            x_hbm.at[pl.ds((b * H + h0) * D, NH * D)], xv[p], sin[p])

    def oout(c, p, e):
        h0 = hc * rows_per_w + c * NH
        return pltpu.make_async_copy(
            ov[p].at[pl.ds(e * NH * K, NH * K)],
            out_hbm.at[pl.ds(((e * B + b) * H + h0) * K, NH * K)],
            sout[p])

    def gather_chunk(x_v, o_v):
        for e in range(E):
            ivs = [idx_v[pl.ds(e * K + kc * 16, 16)]
                   for kc in range(K // 16)]

            def hloop(hl, _):
                base = hl * D
                for kc in range(K // 16):
                    g = plsc.load_gather(x_v, [ivs[kc] + base])
                    o_v[pl.ds((e * NH + hl) * K + kc * 16, 16)] = g
                return 0

            lax.fori_loop(0, NH, hloop, 0)

    xin(0, 0).start()

    def body(cc, _):
        for p in (0, 1):
            c = cc * 2 + p

            @pl.when(c + 1 < nchunks)
            def _():
                xin(c + 1, p ^ 1).start()

            xin(c, p).wait()

            @pl.when(cc >= 1)
            def _():
                for e in range(E):
                    oout(c - 2, p, e).wait()

            gather_chunk(xv[p], ov[p])
            for e in range(E):
                oout(c, p, e).start()
        return 0

    lax.fori_loop(0, nchunks // 2, body, 0)
    for p in (0, 1):
        for e in range(E):
            oout(nchunks - 2 + p, p, e).wait()


def kernel(x, W):
    xt = jnp.swapaxes(x, -1, -2)
    router_logit = jnp.einsum('bdh,eh->bde', xt, W)
    router_logit = jax.nn.softmax(router_logit, axis=-1)
    noise = jax.random.normal(jax.random.key(1234), router_logit.shape,
                              dtype=router_logit.dtype) * 0.001
    v = router_logit + noise
    vt = jnp.swapaxes(v, 1, 2)

    w_bek, i_bek, loss = pl.pallas_call(
        _topk_loss_kernel,
        grid=(B,),
        in_specs=[pl.BlockSpec((1, E, D), lambda b: (b, 0, 0))],
        out_specs=[
            pl.BlockSpec((1, E, K), lambda b: (b, 0, 0)),
            pl.BlockSpec((1, E, K), lambda b: (b, 0, 0)),
            pl.BlockSpec((1, 1), lambda b: (0, 0)),
        ],
        out_shape=[
            jax.ShapeDtypeStruct((B, E, K), jnp.float32),
            jax.ShapeDtypeStruct((B, E, K), jnp.int32),
            jax.ShapeDtypeStruct((1, 1), jnp.float32),
        ],
        scratch_shapes=[pltpu.VMEM((E, D), jnp.float32)],
    )(vt)

    mesh = plsc.VectorSubcoreMesh(core_axis_name="c", subcore_axis_name="s")
    gather = functools.partial(
        pl.kernel,
        mesh=mesh,
        compiler_params=pltpu.CompilerParams(needs_layout_passes=False),
        out_type=jax.ShapeDtypeStruct((E * B * H * K,), jnp.float32),
        scratch_types=[
            pltpu.VMEM((E * K,), jnp.int32),
            pltpu.VMEM((NH * D,), jnp.float32),
            pltpu.VMEM((NH * D,), jnp.float32),
            pltpu.VMEM((E * NH * K,), jnp.float32),
            pltpu.VMEM((E * NH * K,), jnp.float32),
            pltpu.SemaphoreType.DMA,
            pltpu.SemaphoreType.DMA,
            pltpu.SemaphoreType.DMA,
            pltpu.SemaphoreType.DMA,
        ],
    )(_sc_gather)
    tokens = gather(x.reshape(B * H * D),
                    i_bek.reshape(B * E * K)).reshape(E, B, H, K)

    weights = jnp.transpose(w_bek, (1, 0, 2))
    indices = jnp.transpose(i_bek, (1, 0, 2))
    return tokens, weights, indices, loss.reshape(())


# speedup vs baseline: 2.1245x; 1.4527x over previous
import functools

import jax
import jax.numpy as jnp
from jax import lax
from jax.experimental import pallas as pl
from jax.experimental.pallas import tpu as pltpu
from jax.experimental.pallas import tpu_sc as plsc

E = 8
K = 256
D = 2048
H = 2048
B = 4
HT = 256
NH = 8
NW = 32


def _topk_loss_kernel(v_ref, w_ref, i_ref, loss_ref):
    v = v_ref[...]
    iota_d = lax.broadcasted_iota(jnp.int32, (B * E, D), 1)
    iota_k = lax.broadcasted_iota(jnp.int32, (B * E, K), 1)

    def step(kk, carry):
        vals, idxs, work = carry
        m = jnp.max(work, axis=1, keepdims=True)
        am = jnp.min(jnp.where(work == m, iota_d, D), axis=1, keepdims=True)
        vals = jnp.where(iota_k == kk, m, vals)
        idxs = jnp.where(iota_k == kk, am, idxs)
        work = jnp.where(iota_d == am, -jnp.inf, work)
        return vals, idxs, work

    vals0 = jnp.zeros((B * E, K), jnp.float32)
    idxs0 = jnp.zeros((B * E, K), jnp.int32)
    vals, idxs, work = lax.fori_loop(0, K, step, (vals0, idxs0, v))
    w_ref[...] = vals
    i_ref[...] = idxs

    chosen = jnp.where(work == -jnp.inf, 1.0, 0.0).astype(jnp.float32)
    u = (jnp.sum(chosen.reshape(B, E, D), axis=0) * (1.0 / (B * K + 1e-9))
         - (1.0 / E))
    loss_ref[...] = (jnp.sum(u * u) * (1.0 / (E * D))).reshape(1, 1)


def _gather_kernel(x_ref, i_ref, out_ref):
    xb = x_ref[0]
    row = i_ref[0, 0, :].reshape(1, K)
    for dc in range(D // K):
        iota = lax.broadcasted_iota(jnp.int32, (K, K), 0) + dc * K
        p = (iota == row).astype(jnp.float32)
        part = lax.dot_general(
            xb[:, dc * K:(dc + 1) * K], p,
            (((1,), (0,)), ((), ())),
            preferred_element_type=jnp.float32,
        )
        if dc == 0:
            out_ref[0, 0] = part
        else:
            out_ref[0, 0] = out_ref[0, 0] + part


def _sc_gather(x_hbm, idx_hbm, out_hbm, idx_v, x_v0, x_v1, o_v0, o_v1,
               si0, si1, so0, so1):
    wid = lax.axis_index("s") * 2 + lax.axis_index("c")
    b = wid // (NW // B)
    hc = wid % (NW // B)
    rows_per_w = H * B // NW
    nchunks = rows_per_w // NH
    pltpu.sync_copy(idx_hbm.at[pl.ds(b * E * K, E * K)], idx_v)

    xv = (x_v0, x_v1)
    ov = (o_v0, o_v1)
    sin = (si0, si1)
    sout = (so0, so1)

    def xin(c, p):
        h0 = hc * rows_per_w + c * NH
        return pltpu.make_async_copy(
            x_hbm.at[pl.ds((b * H + h0) * D, NH * D)], xv[p], sin[p])

    def oout(c, p, e):
        h0 = hc * rows_per_w + c * NH
        return pltpu.make_async_copy(
            ov[p].at[pl.ds(e * NH * K, NH * K)],
            out_hbm.at[pl.ds(((e * B + b) * H + h0) * K, NH * K)],
            sout[p])

    def gather_chunk(x_v, o_v):
        for e in range(E):
            ivs = [idx_v[pl.ds(e * K + kc * 16, 16)]
                   for kc in range(K // 16)]

            def hloop(hl, _):
                base = hl * D
                for kc in range(K // 16):
                    g = plsc.load_gather(x_v, [ivs[kc] + base])
                    o_v[pl.ds((e * NH + hl) * K + kc * 16, 16)] = g
                return 0

            lax.fori_loop(0, NH, hloop, 0)

    xin(0, 0).start()

    def body(cc, _):
        for p in (0, 1):
            c = cc * 2 + p

            @pl.when(c + 1 < nchunks)
            def _():
                xin(c + 1, p ^ 1).start()

            xin(c, p).wait()

            @pl.when(cc >= 1)
            def _():
                for e in range(E):
                    oout(c - 2, p, e).wait()

            gather_chunk(xv[p], ov[p])
            for e in range(E):
                oout(c, p, e).start()
        return 0

    lax.fori_loop(0, nchunks // 2, body, 0)
    for p in (0, 1):
        for e in range(E):
            oout(nchunks - 2 + p, p, e).wait()


def kernel(x, W):
    router_logit = jnp.einsum('bhd,eh->bde', x, W)
    router_logit = jax.nn.softmax(router_logit, axis=-1)
    noise = jax.random.normal(jax.random.key(1234), router_logit.shape,
                              dtype=router_logit.dtype) * 0.001
    v = router_logit + noise
    vt = jnp.swapaxes(v, 1, 2).reshape(B * E, D)

    w_flat, i_flat, loss = pl.pallas_call(
        _topk_loss_kernel,
        out_shape=[
            jax.ShapeDtypeStruct((B * E, K), jnp.float32),
            jax.ShapeDtypeStruct((B * E, K), jnp.int32),
            jax.ShapeDtypeStruct((1, 1), jnp.float32),
        ],
    )(vt)
    w_bek = w_flat.reshape(B, E, K)
    i_bek = i_flat.reshape(B, E, K)

    mesh = plsc.VectorSubcoreMesh(core_axis_name="c", subcore_axis_name="s")
    gather = functools.partial(
        pl.kernel,
        mesh=mesh,
        compiler_params=pltpu.CompilerParams(needs_layout_passes=False),
        out_type=jax.ShapeDtypeStruct((E * B * H * K,), jnp.float32),
        scratch_types=[
            pltpu.VMEM((E * K,), jnp.int32),
            pltpu.VMEM((NH * D,), jnp.float32),
            pltpu.VMEM((NH * D,), jnp.float32),
            pltpu.VMEM((E * NH * K,), jnp.float32),
            pltpu.VMEM((E * NH * K,), jnp.float32),
            pltpu.SemaphoreType.DMA,
            pltpu.SemaphoreType.DMA,
            pltpu.SemaphoreType.DMA,
            pltpu.SemaphoreType.DMA,
        ],
    )(_sc_gather)
    tokens = gather(x.reshape(B * H * D),
                    i_bek.reshape(B * E * K)).reshape(E, B, H, K)

    weights = jnp.transpose(w_bek, (1, 0, 2))
    indices = jnp.transpose(i_bek, (1, 0, 2))
    return tokens, weights, indices, loss.reshape(())


# speedup vs baseline: 2.1880x; 1.0299x over previous
import functools

import jax
import jax.numpy as jnp
from jax import lax
from jax.experimental import pallas as pl
from jax.experimental.pallas import tpu as pltpu
from jax.experimental.pallas import tpu_sc as plsc

E = 8
K = 256
D = 2048
H = 2048
B = 4
HT = 256
NH = 8
NW = 32


def _topk_loss_kernel(v_ref, w_ref, i_ref, loss_ref):
    v = v_ref[...]
    iota_d = lax.broadcasted_iota(jnp.int32, (B * E, D), 1)
    iota_k = lax.broadcasted_iota(jnp.int32, (B * E, K), 1)

    def step(kk4, carry):
        vals, idxs, work = carry
        for j in range(4):
            kk = kk4 * 4 + j
            m = jnp.max(work, axis=1, keepdims=True)
            am = jnp.min(jnp.where(work == m, iota_d, D), axis=1,
                         keepdims=True)
            vals = jnp.where(iota_k == kk, m, vals)
            idxs = jnp.where(iota_k == kk, am, idxs)
            work = jnp.where(iota_d == am, -jnp.inf, work)
        return vals, idxs, work

    vals0 = jnp.zeros((B * E, K), jnp.float32)
    idxs0 = jnp.zeros((B * E, K), jnp.int32)
    vals, idxs, work = lax.fori_loop(0, K // 4, step, (vals0, idxs0, v))
    w_ref[...] = vals
    i_ref[...] = idxs

    chosen = jnp.where(work == -jnp.inf, 1.0, 0.0).astype(jnp.float32)
    u = (jnp.sum(chosen.reshape(B, E, D), axis=0) * (1.0 / (B * K + 1e-9))
         - (1.0 / E))
    loss_ref[...] = (jnp.sum(u * u) * (1.0 / (E * D))).reshape(1, 1)


def _gather_kernel(x_ref, i_ref, out_ref):
    xb = x_ref[0]
    row = i_ref[0, 0, :].reshape(1, K)
    for dc in range(D // K):
        iota = lax.broadcasted_iota(jnp.int32, (K, K), 0) + dc * K
        p = (iota == row).astype(jnp.float32)
        part = lax.dot_general(
            xb[:, dc * K:(dc + 1) * K], p,
            (((1,), (0,)), ((), ())),
            preferred_element_type=jnp.float32,
        )
        if dc == 0:
            out_ref[0, 0] = part
        else:
            out_ref[0, 0] = out_ref[0, 0] + part


def _sc_gather(x_hbm, idx_hbm, out_hbm, idx_v, x_v0, x_v1, o_v0, o_v1,
               si0, si1, so0, so1):
    wid = lax.axis_index("s") * 2 + lax.axis_index("c")
    b = wid // (NW // B)
    hc = wid % (NW // B)
    rows_per_w = H * B // NW
    nchunks = rows_per_w // NH
    pltpu.sync_copy(idx_hbm.at[pl.ds(b * E * K, E * K)], idx_v)

    xv = (x_v0, x_v1)
    ov = (o_v0, o_v1)
    sin = (si0, si1)
    sout = (so0, so1)

    def xin(c, p):
        h0 = hc * rows_per_w + c * NH
        return pltpu.make_async_copy(
            x_hbm.at[pl.ds((b * H + h0) * D, NH * D)], xv[p], sin[p])

    def oout(c, p, e):
        h0 = hc * rows_per_w + c * NH
        return pltpu.make_async_copy(
            ov[p].at[pl.ds(e * NH * K, NH * K)],
            out_hbm.at[pl.ds(((e * B + b) * H + h0) * K, NH * K)],
            sout[p])

    def gather_chunk(x_v, o_v):
        for e in range(E):
            ivs = [idx_v[pl.ds(e * K + kc * 16, 16)]
                   for kc in range(K // 16)]

            def hloop(hl, _):
                base = hl * D
                for kc in range(K // 16):
                    g = plsc.load_gather(x_v, [ivs[kc] + base])
                    o_v[pl.ds((e * NH + hl) * K + kc * 16, 16)] = g
                return 0

            lax.fori_loop(0, NH, hloop, 0)

    xin(0, 0).start()

    def body(cc, _):
        for p in (0, 1):
            c = cc * 2 + p

            @pl.when(c + 1 < nchunks)
            def _():
                xin(c + 1, p ^ 1).start()

            xin(c, p).wait()

            @pl.when(cc >= 1)
            def _():
                for e in range(E):
                    oout(c - 2, p, e).wait()

            gather_chunk(xv[p], ov[p])
            for e in range(E):
                oout(c, p, e).start()
        return 0

    lax.fori_loop(0, nchunks // 2, body, 0)
    for p in (0, 1):
        for e in range(E):
            oout(nchunks - 2 + p, p, e).wait()


def kernel(x, W):
    router_logit = jnp.einsum('bhd,eh->bde', x, W)
    router_logit = jax.nn.softmax(router_logit, axis=-1)
    noise = jax.random.normal(jax.random.key(1234), router_logit.shape,
                              dtype=router_logit.dtype) * 0.001
    v = router_logit + noise
    vt = jnp.swapaxes(v, 1, 2).reshape(B * E, D)

    w_flat, i_flat, loss = pl.pallas_call(
        _topk_loss_kernel,
        out_shape=[
            jax.ShapeDtypeStruct((B * E, K), jnp.float32),
            jax.ShapeDtypeStruct((B * E, K), jnp.int32),
            jax.ShapeDtypeStruct((1, 1), jnp.float32),
        ],
    )(vt)
    w_bek = w_flat.reshape(B, E, K)
    i_bek = i_flat.reshape(B, E, K)

    mesh = plsc.VectorSubcoreMesh(core_axis_name="c", subcore_axis_name="s")
    gather = functools.partial(
        pl.kernel,
        mesh=mesh,
        compiler_params=pltpu.CompilerParams(needs_layout_passes=False),
        out_type=jax.ShapeDtypeStruct((E * B * H * K,), jnp.float32),
        scratch_types=[
            pltpu.VMEM((E * K,), jnp.int32),
            pltpu.VMEM((NH * D,), jnp.float32),
            pltpu.VMEM((NH * D,), jnp.float32),
            pltpu.VMEM((E * NH * K,), jnp.float32),
            pltpu.VMEM((E * NH * K,), jnp.float32),
            pltpu.SemaphoreType.DMA,
            pltpu.SemaphoreType.DMA,
            pltpu.SemaphoreType.DMA,
            pltpu.SemaphoreType.DMA,
        ],
    )(_sc_gather)
    tokens = gather(x.reshape(B * H * D),
                    i_bek.reshape(B * E * K)).reshape(E, B, H, K)

    weights = jnp.transpose(w_bek, (1, 0, 2))
    indices = jnp.transpose(i_bek, (1, 0, 2))
    return tokens, weights, indices, loss.reshape(())


# speedup vs baseline: 2.2070x; 1.0087x over previous
import functools

import jax
import jax.numpy as jnp
from jax import lax
from jax.experimental import pallas as pl
from jax.experimental.pallas import tpu as pltpu
from jax.experimental.pallas import tpu_sc as plsc

E = 8
K = 256
D = 2048
H = 2048
B = 4
HT = 256
NH = 8
NW = 32


def _topk_loss_kernel(v_ref, w_ref, i_ref, loss_ref):
    v = v_ref[...]
    iota_d = lax.broadcasted_iota(jnp.int32, (B * E, D), 1)
    iota_k = lax.broadcasted_iota(jnp.int32, (B * E, K), 1)

    def step(kk4, carry):
        vals, idxs, work = carry
        for j in range(8):
            kk = kk4 * 8 + j
            m = jnp.max(work, axis=1, keepdims=True)
            am = jnp.min(jnp.where(work == m, iota_d, D), axis=1,
                         keepdims=True)
            vals = jnp.where(iota_k == kk, m, vals)
            idxs = jnp.where(iota_k == kk, am, idxs)
            work = jnp.where(iota_d == am, -jnp.inf, work)
        return vals, idxs, work

    vals0 = jnp.zeros((B * E, K), jnp.float32)
    idxs0 = jnp.zeros((B * E, K), jnp.int32)
    vals, idxs, work = lax.fori_loop(0, K // 8, step, (vals0, idxs0, v))
    w_ref[...] = vals
    i_ref[...] = idxs

    chosen = jnp.where(work == -jnp.inf, 1.0, 0.0).astype(jnp.float32)
    u = (jnp.sum(chosen.reshape(B, E, D), axis=0) * (1.0 / (B * K + 1e-9))
         - (1.0 / E))
    loss_ref[...] = (jnp.sum(u * u) * (1.0 / (E * D))).reshape(1, 1)


def _gather_kernel(x_ref, i_ref, out_ref):
    xb = x_ref[0]
    row = i_ref[0, 0, :].reshape(1, K)
    for dc in range(D // K):
        iota = lax.broadcasted_iota(jnp.int32, (K, K), 0) + dc * K
        p = (iota == row).astype(jnp.float32)
        part = lax.dot_general(
            xb[:, dc * K:(dc + 1) * K], p,
            (((1,), (0,)), ((), ())),
            preferred_element_type=jnp.float32,
        )
        if dc == 0:
            out_ref[0, 0] = part
        else:
            out_ref[0, 0] = out_ref[0, 0] + part


def _sc_gather(x_hbm, idx_hbm, out_hbm, idx_v, x_v0, x_v1, o_v0, o_v1,
               si0, si1, so0, so1):
    wid = lax.axis_index("s") * 2 + lax.axis_index("c")
    b = wid // (NW // B)
    hc = wid % (NW // B)
    rows_per_w = H * B // NW
    nchunks = rows_per_w // NH
    pltpu.sync_copy(idx_hbm.at[pl.ds(b * E * K, E * K)], idx_v)

    xv = (x_v0, x_v1)
    ov = (o_v0, o_v1)
    sin = (si0, si1)
    sout = (so0, so1)

    def xin(c, p):
        h0 = hc * rows_per_w + c * NH
        return pltpu.make_async_copy(
            x_hbm.at[pl.ds((b * H + h0) * D, NH * D)], xv[p], sin[p])

    def oout(c, p, e):
        h0 = hc * rows_per_w + c * NH
        return pltpu.make_async_copy(
            ov[p].at[pl.ds(e * NH * K, NH * K)],
            out_hbm.at[pl.ds(((e * B + b) * H + h0) * K, NH * K)],
            sout[p])

    def gather_chunk(x_v, o_v):
        for e in range(E):
            ivs = [idx_v[pl.ds(e * K + kc * 16, 16)]
                   for kc in range(K // 16)]

            def hloop(hl, _):
                base = hl * D
                for kc in range(K // 16):
                    g = plsc.load_gather(x_v, [ivs[kc] + base])
                    o_v[pl.ds((e * NH + hl) * K + kc * 16, 16)] = g
                return 0

            lax.fori_loop(0, NH, hloop, 0)

    xin(0, 0).start()

    def body(cc, _):
        for p in (0, 1):
            c = cc * 2 + p

            @pl.when(c + 1 < nchunks)
            def _():
                xin(c + 1, p ^ 1).start()

            xin(c, p).wait()

            @pl.when(cc >= 1)
            def _():
                for e in range(E):
                    oout(c - 2, p, e).wait()

            gather_chunk(xv[p], ov[p])
            for e in range(E):
                oout(c, p, e).start()
        return 0

    lax.fori_loop(0, nchunks // 2, body, 0)
    for p in (0, 1):
        for e in range(E):
            oout(nchunks - 2 + p, p, e).wait()


def kernel(x, W):
    router_logit = jnp.einsum('bhd,eh->bde', x, W)
    router_logit = jax.nn.softmax(router_logit, axis=-1)
    noise = jax.random.normal(jax.random.key(1234), router_logit.shape,
                              dtype=router_logit.dtype) * 0.001
    v = router_logit + noise
    vt = jnp.swapaxes(v, 1, 2).reshape(B * E, D)

    w_flat, i_flat, loss = pl.pallas_call(
        _topk_loss_kernel,
        out_shape=[
            jax.ShapeDtypeStruct((B * E, K), jnp.float32),
            jax.ShapeDtypeStruct((B * E, K), jnp.int32),
            jax.ShapeDtypeStruct((1, 1), jnp.float32),
        ],
    )(vt)
    w_bek = w_flat.reshape(B, E, K)
    i_bek = i_flat.reshape(B, E, K)

    mesh = plsc.VectorSubcoreMesh(core_axis_name="c", subcore_axis_name="s")
    gather = functools.partial(
        pl.kernel,
        mesh=mesh,
        compiler_params=pltpu.CompilerParams(needs_layout_passes=False),
        out_type=jax.ShapeDtypeStruct((E * B * H * K,), jnp.float32),
        scratch_types=[
            pltpu.VMEM((E * K,), jnp.int32),
            pltpu.VMEM((NH * D,), jnp.float32),
            pltpu.VMEM((NH * D,), jnp.float32),
            pltpu.VMEM((E * NH * K,), jnp.float32),
            pltpu.VMEM((E * NH * K,), jnp.float32),
            pltpu.SemaphoreType.DMA,
            pltpu.SemaphoreType.DMA,
            pltpu.SemaphoreType.DMA,
            pltpu.SemaphoreType.DMA,
        ],
    )(_sc_gather)
    tokens = gather(x.reshape(B * H * D),
                    i_bek.reshape(B * E * K)).reshape(E, B, H, K)

    weights = jnp.transpose(w_bek, (1, 0, 2))
    indices = jnp.transpose(i_bek, (1, 0, 2))
    return tokens, weights, indices, loss.reshape(())
